# Initial kernel scaffold; baseline (speedup 1.0000x reference)
import jax
import jax.numpy as jnp
from jax.experimental import pallas as pl


def kernel(x_user, x_movie, edge_index_um, edge_index_mu, edge_label_index, Wl1_um, Wr1_um, b1_um, Wl1_mu, Wr1_mu, b1_mu, Wl2_um, Wr2_um, b2_um, Wl2_mu, Wr2_mu, b2_mu):
    raise NotImplementedError("write your pallas kernel here")


# speedup vs baseline: 4.5875x; 4.5875x over previous
import functools

import jax
import jax.numpy as jnp
from jax import lax
from jax.experimental import pallas as pl
from jax.experimental.pallas import tpu as pltpu
from jax.experimental.pallas import tpu_sc as plsc

N = 10000
NPAD = 10240
D = 128
H = 64
W = 128
E = 320000
LBL = 100000

NC = 2
NS = 16
NW = NC * NS
LANE = 16

EPT = E // NW
K = 200
NCHUNK = EPT // K
RPT = NPAD // NS

K2 = 400
NCH_DEC = LBL // K2
DEC_PER_TILE = (NCH_DEC + NW - 1) // NW


def _sc_mesh():
    return plsc.VectorSubcoreMesh(core_axis_name="c", subcore_axis_name="s",
                                  num_cores=NC, num_subcores=NS)


def _make_segsum():
    out_type = [jax.ShapeDtypeStruct((NC, NPAD, W), jnp.float32),
                jax.ShapeDtypeStruct((NC, NPAD, W), jnp.float32)]
    scratch = [
        pltpu.VMEM((K,), jnp.int32),
        pltpu.VMEM((K,), jnp.int32),
        pltpu.VMEM((K, W), jnp.float32),
        pltpu.VMEM_SHARED((NPAD, W), jnp.float32),
        pltpu.SemaphoreType.DMA,
    ]

    def body(t_a, src_a, dst_a, t_b, src_b, dst_b, zrow,
             acc_a_o, acc_b_o, idx_s, idx_d, rows_v, acc_sh, sem):
        cid = lax.axis_index("c")
        sid = lax.axis_index("s")
        wid = sid * NC + cid
        rbase = pl.multiple_of(sid * RPT, 8)

        def one_type(tbl, src, dst, acc_o):
            pltpu.sync_copy(zrow.at[pl.ds(rbase, RPT)],
                            acc_sh.at[pl.ds(rbase, RPT)])
            plsc.subcore_barrier()

            ebase = wid * EPT

            def chunk(ci, carry):
                e0 = pl.multiple_of(ebase + ci * K, 8)
                pltpu.sync_copy(src.at[pl.ds(e0, K)], idx_s)
                pltpu.sync_copy(dst.at[pl.ds(e0, K)], idx_d)
                pltpu.async_copy(tbl.at[idx_s], rows_v, sem).wait()
                pltpu.sync_copy(rows_v, acc_sh.at[idx_d], add=True)
                return carry

            lax.fori_loop(0, NCHUNK, chunk, 0)
            plsc.subcore_barrier()

            pltpu.sync_copy(acc_sh.at[pl.ds(rbase, RPT)],
                            acc_o.at[cid, pl.ds(rbase, RPT)])
            plsc.subcore_barrier()

        one_type(t_a, src_a, dst_a, acc_a_o)
        one_type(t_b, src_b, dst_b, acc_b_o)

    return pl.kernel(body, out_type=out_type, mesh=_sc_mesh(),
                     scratch_types=scratch)


def _make_decoder():
    out_type = [jax.ShapeDtypeStruct((LBL, W), jnp.float32),
                jax.ShapeDtypeStruct((LBL, W), jnp.float32)]
    scratch = [
        pltpu.VMEM((K2,), jnp.int32),
        pltpu.VMEM((K2,), jnp.int32),
        pltpu.VMEM((K2, W), jnp.float32),
        pltpu.VMEM((K2, W), jnp.float32),
        pltpu.SemaphoreType.DMA,
    ]

    def body(h_u, h_m, e_u, e_m, ug_o, mg_o, idx_u, idx_m, u_rows, m_rows,
             sem):
        cid = lax.axis_index("c")
        sid = lax.axis_index("s")
        wid = sid * NC + cid

        def per_j(j, carry):
            ch = j * NW + wid

            @pl.when(ch < NCH_DEC)
            def _():
                base = pl.multiple_of(ch * K2, 8)
                pltpu.sync_copy(e_u.at[pl.ds(base, K2)], idx_u)
                pltpu.sync_copy(e_m.at[pl.ds(base, K2)], idx_m)
                pltpu.async_copy(h_u.at[idx_u], u_rows, sem).wait()
                pltpu.async_copy(h_m.at[idx_m], m_rows, sem).wait()
                pltpu.sync_copy(u_rows, ug_o.at[pl.ds(base, K2)])
                pltpu.sync_copy(m_rows, mg_o.at[pl.ds(base, K2)])

            return carry

        lax.fori_loop(0, DEC_PER_TILE, per_j, 0)

    return pl.kernel(body, out_type=out_type, mesh=_sc_mesh(),
                     scratch_types=scratch)


def _tcdot_body(ug_r, mg_r, out_r):
    u = ug_r[...]
    m = mg_r[...]
    out_r[...] = jnp.sum(u[:, :H] * m[:, :H], axis=1, keepdims=True)


def _tcdot(ug, mg):
    RL = 2000
    return pl.pallas_call(
        _tcdot_body,
        grid=(LBL // RL,),
        in_specs=[pl.BlockSpec((RL, W), lambda i: (i, 0)),
                  pl.BlockSpec((RL, W), lambda i: (i, 0))],
        out_specs=pl.BlockSpec((RL, 1), lambda i: (i, 0)),
        out_shape=jax.ShapeDtypeStruct((LBL, 1), jnp.float32),
    )(ug, mg)


R = 1000
G = N // R

_mm = functools.partial(jnp.dot, precision="highest",
                        preferred_element_type=jnp.float32)


def _with_cols(main, extra_col=None):
    cols = [main]
    used = H
    if extra_col is not None:
        cols.append(extra_col)
        used += 1
    cols.append(jnp.zeros((main.shape[0], W - used), jnp.float32))
    return jnp.concatenate(cols, axis=1)


def _tc1_body(xu_r, xm_r, wlu_r, wru_r, wlm_r, wrm_r,
              tlu_r, yru_r, tlm_r, yrm_r):
    xu = xu_r[...]
    xm = xm_r[...]
    one = jnp.ones((R, 1), jnp.float32)
    tlu_r[...] = _with_cols(_mm(xu, wlu_r[...]), one)
    yru_r[...] = _mm(xu, wru_r[...])
    tlm_r[...] = _with_cols(_mm(xm, wlm_r[...]), one)
    yrm_r[...] = _mm(xm, wrm_r[...])


def _tc1(xu, xm, wl1_um, wr1_mu, wl1_mu, wr1_um):
    fs = jnp.float32
    return pl.pallas_call(
        _tc1_body,
        grid=(G,),
        in_specs=[
            pl.BlockSpec((R, D), lambda i: (i, 0)),
            pl.BlockSpec((R, D), lambda i: (i, 0)),
            pl.BlockSpec((D, H), lambda i: (0, 0)),
            pl.BlockSpec((D, H), lambda i: (0, 0)),
            pl.BlockSpec((D, H), lambda i: (0, 0)),
            pl.BlockSpec((D, H), lambda i: (0, 0)),
        ],
        out_specs=[
            pl.BlockSpec((R, W), lambda i: (i, 0)),
            pl.BlockSpec((R, H), lambda i: (i, 0)),
            pl.BlockSpec((R, W), lambda i: (i, 0)),
            pl.BlockSpec((R, H), lambda i: (i, 0)),
        ],
        out_shape=[
            jax.ShapeDtypeStruct((N, W), fs),
            jax.ShapeDtypeStruct((N, H), fs),
            jax.ShapeDtypeStruct((N, W), fs),
            jax.ShapeDtypeStruct((N, H), fs),
        ],
    )(xu, xm, wl1_um, wr1_mu, wl1_mu, wr1_um)


def _tc2_body(am_r, au_r, yrm_r, yru_r, b1um_r, b1mu_r,
              wl2um_r, wr2um_r, wl2mu_r, wr2mu_r, b2um_r, b2mu_r,
              tl2u_r, tl2m_r, zmx_r, zux_r):
    am = am_r[...]
    au = au_r[...]
    sm = am[0] + am[1]
    su = au[0] + au[1]
    icm = 1.0 / jnp.maximum(sm[:, H:H + 1], 1.0)
    icu = 1.0 / jnp.maximum(su[:, H:H + 1], 1.0)
    hm = jnp.maximum(sm[:, :H] * icm + yrm_r[...] + b1um_r[...], 0.0)
    hu = jnp.maximum(su[:, :H] * icu + yru_r[...] + b1mu_r[...], 0.0)
    tl2u_r[...] = _with_cols(_mm(hu, wl2um_r[...]))
    tl2m_r[...] = _with_cols(_mm(hm, wl2mu_r[...]))
    zmx_r[...] = _with_cols(_mm(hm, wr2um_r[...]) + b2um_r[...], icm)
    zux_r[...] = _with_cols(_mm(hu, wr2mu_r[...]) + b2mu_r[...], icu)


def _tc2(acc_m, acc_u, yrm, yru, b1um, b1mu,
         wl2um, wr2um, wl2mu, wr2mu, b2um, b2mu):
    fs = jnp.float32
    part = pl.BlockSpec((NC, R, W), lambda i: (0, i, 0))
    row = pl.BlockSpec((R, H), lambda i: (i, 0))
    roww = pl.BlockSpec((R, W), lambda i: (i, 0))
    w = pl.BlockSpec((H, H), lambda i: (0, 0))
    b = pl.BlockSpec((1, H), lambda i: (0, 0))
    return pl.pallas_call(
        _tc2_body,
        grid=(G,),
        in_specs=[part, part, row, row, b, b, w, w, w, w, b, b],
        out_specs=[roww] * 4,
        out_shape=[jax.ShapeDtypeStruct((N, W), fs)] * 4,
    )(acc_m, acc_u, yrm, yru, b1um, b1mu,
      wl2um, wr2um, wl2mu, wr2mu, b2um, b2mu)


def _tc3_body(am_r, au_r, zmx_r, zux_r, hm2_r, hu2_r):
    am = am_r[...]
    au = au_r[...]
    sm = am[0] + am[1]
    su = au[0] + au[1]
    zmx = zmx_r[...]
    zux = zux_r[...]
    hm2 = sm[:, :H] * zmx[:, H:H + 1] + zmx[:, :H]
    hu2 = su[:, :H] * zux[:, H:H + 1] + zux[:, :H]
    hm2_r[...] = _with_cols(hm2)
    hu2_r[...] = _with_cols(hu2)


def _tc3(acc2_m, acc2_u, zmx, zux):
    fs = jnp.float32
    part = pl.BlockSpec((NC, R, W), lambda i: (0, i, 0))
    roww = pl.BlockSpec((R, W), lambda i: (i, 0))
    return pl.pallas_call(
        _tc3_body,
        grid=(G,),
        in_specs=[part, part, roww, roww],
        out_specs=[roww] * 2,
        out_shape=[jax.ShapeDtypeStruct((N, W), fs)] * 2,
    )(acc2_m, acc2_u, zmx, zux)


def kernel(x_user, x_movie, edge_index_um, edge_index_mu, edge_label_index,
           Wl1_um, Wr1_um, b1_um, Wl1_mu, Wr1_mu, b1_mu,
           Wl2_um, Wr2_um, b2_um, Wl2_mu, Wr2_mu, b2_mu):
    src_um, dst_um = edge_index_um[0], edge_index_um[1]
    src_mu, dst_mu = edge_index_mu[0], edge_index_mu[1]
    e_u, e_m = edge_label_index[0], edge_label_index[1]

    zrow = jnp.zeros((NPAD, W), jnp.float32)

    b1um = b1_um.reshape(1, H)
    b1mu = b1_mu.reshape(1, H)
    b2um = b2_um.reshape(1, H)
    b2mu = b2_mu.reshape(1, H)

    tlu, yru, tlm, yrm = _tc1(x_user, x_movie, Wl1_um, Wr1_mu, Wl1_mu,
                              Wr1_um)

    seg = _make_segsum()
    acc_m, acc_u = seg(tlu, src_um, dst_um, tlm, src_mu, dst_mu, zrow)

    tl2u, tl2m, zmx, zux = _tc2(acc_m, acc_u, yrm, yru, b1um, b1mu,
                                Wl2_um, Wr2_um, Wl2_mu, Wr2_mu, b2um, b2mu)

    seg2 = _make_segsum()
    acc2_m, acc2_u = seg2(tl2u, src_um, dst_um, tl2m, src_mu, dst_mu, zrow)

    hm2, hu2 = _tc3(acc2_m, acc2_u, zmx, zux)

    dec = _make_decoder()
    ug, mg = dec(hu2, hm2, e_u, e_m)
    return _tcdot(ug, mg).reshape(LBL)


# speedup vs baseline: 4.7226x; 1.0295x over previous
import functools

import jax
import jax.numpy as jnp
from jax import lax
from jax.experimental import pallas as pl
from jax.experimental.pallas import tpu as pltpu
from jax.experimental.pallas import tpu_sc as plsc

N = 10000
NPAD = 10240
D = 128
H = 64
W = 128
E = 320000
LBL = 100000

NC = 2
NS = 16
NW = NC * NS
LANE = 16

EPT = E // NW
K = 100
NCHUNK = EPT // K
RPT = NPAD // NS

K2 = 400
NCH_DEC = LBL // K2
DEC_PER_TILE = (NCH_DEC + NW - 1) // NW


def _sc_mesh():
    return plsc.VectorSubcoreMesh(core_axis_name="c", subcore_axis_name="s",
                                  num_cores=NC, num_subcores=NS)


def _make_segsum():
    out_type = [jax.ShapeDtypeStruct((NC, NPAD, W), jnp.float32),
                jax.ShapeDtypeStruct((NC, NPAD, W), jnp.float32)]
    scratch = [
        pltpu.VMEM((EPT,), jnp.int32),
        pltpu.VMEM((NCHUNK, K), jnp.int32),
        pltpu.VMEM((K, W), jnp.float32),
        pltpu.VMEM_SHARED((NPAD, W), jnp.float32),
        pltpu.SemaphoreType.DMA,
    ]

    def body(t_a, src_a, dst_a, t_b, src_b, dst_b, zrow,
             acc_a_o, acc_b_o, idx_s, idx_d, rows_v, acc_sh, sem):
        cid = lax.axis_index("c")
        sid = lax.axis_index("s")
        wid = sid * NC + cid
        rbase = pl.multiple_of(sid * RPT, 8)

        def one_type(tbl, src, dst, acc_o):
            pltpu.sync_copy(
                src.at[pl.ds(pl.multiple_of(wid * EPT, 8), EPT)], idx_s)
            pltpu.sync_copy(dst.at[wid], idx_d)
            pltpu.sync_copy(zrow.at[pl.ds(rbase, RPT)],
                            acc_sh.at[pl.ds(rbase, RPT)])
            plsc.subcore_barrier()

            def chunk(ci, carry):
                sidx = idx_s.at[pl.ds(pl.multiple_of(ci * K, 8), K)]
                pltpu.async_copy(tbl.at[sidx], rows_v, sem).wait()
                pltpu.sync_copy(rows_v, acc_sh.at[idx_d.at[ci]], add=True)
                return carry

            lax.fori_loop(0, NCHUNK, chunk, 0)
            plsc.subcore_barrier()

            pltpu.sync_copy(acc_sh.at[pl.ds(rbase, RPT)],
                            acc_o.at[cid, pl.ds(rbase, RPT)])
            plsc.subcore_barrier()

        one_type(t_a, src_a, dst_a, acc_a_o)
        one_type(t_b, src_b, dst_b, acc_b_o)

    return pl.kernel(body, out_type=out_type, mesh=_sc_mesh(),
                     scratch_types=scratch)


def _make_decoder():
    out_type = [jax.ShapeDtypeStruct((LBL, W), jnp.float32),
                jax.ShapeDtypeStruct((LBL, W), jnp.float32)]
    scratch = [
        pltpu.VMEM((K2,), jnp.int32),
        pltpu.VMEM((K2,), jnp.int32),
        pltpu.VMEM((K2, W), jnp.float32),
        pltpu.VMEM((K2, W), jnp.float32),
        pltpu.SemaphoreType.DMA,
    ]

    def body(h_u, h_m, e_u, e_m, ug_o, mg_o, idx_u, idx_m, u_rows, m_rows,
             sem):
        cid = lax.axis_index("c")
        sid = lax.axis_index("s")
        wid = sid * NC + cid

        def per_j(j, carry):
            ch = j * NW + wid

            @pl.when(ch < NCH_DEC)
            def _():
                base = pl.multiple_of(ch * K2, 8)
                pltpu.sync_copy(e_u.at[pl.ds(base, K2)], idx_u)
                pltpu.sync_copy(e_m.at[pl.ds(base, K2)], idx_m)
                pltpu.async_copy(h_u.at[idx_u], u_rows, sem).wait()
                pltpu.async_copy(h_m.at[idx_m], m_rows, sem).wait()
                pltpu.sync_copy(u_rows, ug_o.at[pl.ds(base, K2)])
                pltpu.sync_copy(m_rows, mg_o.at[pl.ds(base, K2)])

            return carry

        lax.fori_loop(0, DEC_PER_TILE, per_j, 0)

    return pl.kernel(body, out_type=out_type, mesh=_sc_mesh(),
                     scratch_types=scratch)


def _tcdot_body(ug_r, mg_r, out_r):
    u = ug_r[...]
    m = mg_r[...]
    out_r[...] = jnp.sum(u[:, :H] * m[:, :H], axis=1, keepdims=True)


def _tcdot(ug, mg):
    RL = 2000
    return pl.pallas_call(
        _tcdot_body,
        grid=(LBL // RL,),
        in_specs=[pl.BlockSpec((RL, W), lambda i: (i, 0)),
                  pl.BlockSpec((RL, W), lambda i: (i, 0))],
        out_specs=pl.BlockSpec((RL, 1), lambda i: (i, 0)),
        out_shape=jax.ShapeDtypeStruct((LBL, 1), jnp.float32),
    )(ug, mg)


R = 1000
G = N // R

_mm = functools.partial(jnp.dot, precision="highest",
                        preferred_element_type=jnp.float32)


def _with_cols(main, extra_col=None):
    cols = [main]
    used = H
    if extra_col is not None:
        cols.append(extra_col)
        used += 1
    cols.append(jnp.zeros((main.shape[0], W - used), jnp.float32))
    return jnp.concatenate(cols, axis=1)


def _tc1_body(xu_r, xm_r, wlu_r, wru_r, wlm_r, wrm_r,
              tlu_r, yru_r, tlm_r, yrm_r):
    xu = xu_r[...]
    xm = xm_r[...]
    one = jnp.ones((R, 1), jnp.float32)
    tlu_r[...] = _with_cols(_mm(xu, wlu_r[...]), one)
    yru_r[...] = _mm(xu, wru_r[...])
    tlm_r[...] = _with_cols(_mm(xm, wlm_r[...]), one)
    yrm_r[...] = _mm(xm, wrm_r[...])


def _tc1(xu, xm, wl1_um, wr1_mu, wl1_mu, wr1_um):
    fs = jnp.float32
    return pl.pallas_call(
        _tc1_body,
        grid=(G,),
        in_specs=[
            pl.BlockSpec((R, D), lambda i: (i, 0)),
            pl.BlockSpec((R, D), lambda i: (i, 0)),
            pl.BlockSpec((D, H), lambda i: (0, 0)),
            pl.BlockSpec((D, H), lambda i: (0, 0)),
            pl.BlockSpec((D, H), lambda i: (0, 0)),
            pl.BlockSpec((D, H), lambda i: (0, 0)),
        ],
        out_specs=[
            pl.BlockSpec((R, W), lambda i: (i, 0)),
            pl.BlockSpec((R, H), lambda i: (i, 0)),
            pl.BlockSpec((R, W), lambda i: (i, 0)),
            pl.BlockSpec((R, H), lambda i: (i, 0)),
        ],
        out_shape=[
            jax.ShapeDtypeStruct((N, W), fs),
            jax.ShapeDtypeStruct((N, H), fs),
            jax.ShapeDtypeStruct((N, W), fs),
            jax.ShapeDtypeStruct((N, H), fs),
        ],
    )(xu, xm, wl1_um, wr1_mu, wl1_mu, wr1_um)


def _tc2_body(am_r, au_r, yrm_r, yru_r, b1um_r, b1mu_r,
              wl2um_r, wr2um_r, wl2mu_r, wr2mu_r, b2um_r, b2mu_r,
              tl2u_r, tl2m_r, zmx_r, zux_r):
    am = am_r[...]
    au = au_r[...]
    sm = am[0] + am[1]
    su = au[0] + au[1]
    icm = 1.0 / jnp.maximum(sm[:, H:H + 1], 1.0)
    icu = 1.0 / jnp.maximum(su[:, H:H + 1], 1.0)
    hm = jnp.maximum(sm[:, :H] * icm + yrm_r[...] + b1um_r[...], 0.0)
    hu = jnp.maximum(su[:, :H] * icu + yru_r[...] + b1mu_r[...], 0.0)
    tl2u_r[...] = _with_cols(_mm(hu, wl2um_r[...]))
    tl2m_r[...] = _with_cols(_mm(hm, wl2mu_r[...]))
    zmx_r[...] = _with_cols(_mm(hm, wr2um_r[...]) + b2um_r[...], icm)
    zux_r[...] = _with_cols(_mm(hu, wr2mu_r[...]) + b2mu_r[...], icu)


def _tc2(acc_m, acc_u, yrm, yru, b1um, b1mu,
         wl2um, wr2um, wl2mu, wr2mu, b2um, b2mu):
    fs = jnp.float32
    part = pl.BlockSpec((NC, R, W), lambda i: (0, i, 0))
    row = pl.BlockSpec((R, H), lambda i: (i, 0))
    roww = pl.BlockSpec((R, W), lambda i: (i, 0))
    w = pl.BlockSpec((H, H), lambda i: (0, 0))
    b = pl.BlockSpec((1, H), lambda i: (0, 0))
    return pl.pallas_call(
        _tc2_body,
        grid=(G,),
        in_specs=[part, part, row, row, b, b, w, w, w, w, b, b],
        out_specs=[roww] * 4,
        out_shape=[jax.ShapeDtypeStruct((N, W), fs)] * 4,
    )(acc_m, acc_u, yrm, yru, b1um, b1mu,
      wl2um, wr2um, wl2mu, wr2mu, b2um, b2mu)


def _tc3_body(am_r, au_r, zmx_r, zux_r, hm2_r, hu2_r):
    am = am_r[...]
    au = au_r[...]
    sm = am[0] + am[1]
    su = au[0] + au[1]
    zmx = zmx_r[...]
    zux = zux_r[...]
    hm2 = sm[:, :H] * zmx[:, H:H + 1] + zmx[:, :H]
    hu2 = su[:, :H] * zux[:, H:H + 1] + zux[:, :H]
    hm2_r[...] = _with_cols(hm2)
    hu2_r[...] = _with_cols(hu2)


def _tc3(acc2_m, acc2_u, zmx, zux):
    fs = jnp.float32
    part = pl.BlockSpec((NC, R, W), lambda i: (0, i, 0))
    roww = pl.BlockSpec((R, W), lambda i: (i, 0))
    return pl.pallas_call(
        _tc3_body,
        grid=(G,),
        in_specs=[part, part, roww, roww],
        out_specs=[roww] * 2,
        out_shape=[jax.ShapeDtypeStruct((N, W), fs)] * 2,
    )(acc2_m, acc2_u, zmx, zux)


def kernel(x_user, x_movie, edge_index_um, edge_index_mu, edge_label_index,
           Wl1_um, Wr1_um, b1_um, Wl1_mu, Wr1_mu, b1_mu,
           Wl2_um, Wr2_um, b2_um, Wl2_mu, Wr2_mu, b2_mu):
    src_um = edge_index_um[0]
    dst_um = edge_index_um[1].reshape(NW, NCHUNK, K)
    src_mu = edge_index_mu[0]
    dst_mu = edge_index_mu[1].reshape(NW, NCHUNK, K)
    e_u, e_m = edge_label_index[0], edge_label_index[1]

    zrow = jnp.zeros((NPAD, W), jnp.float32)

    b1um = b1_um.reshape(1, H)
    b1mu = b1_mu.reshape(1, H)
    b2um = b2_um.reshape(1, H)
    b2mu = b2_mu.reshape(1, H)

    tlu, yru, tlm, yrm = _tc1(x_user, x_movie, Wl1_um, Wr1_mu, Wl1_mu,
                              Wr1_um)

    seg = _make_segsum()
    acc_m, acc_u = seg(tlu, src_um, dst_um, tlm, src_mu, dst_mu, zrow)

    tl2u, tl2m, zmx, zux = _tc2(acc_m, acc_u, yrm, yru, b1um, b1mu,
                                Wl2_um, Wr2_um, Wl2_mu, Wr2_mu, b2um, b2mu)

    seg2 = _make_segsum()
    acc2_m, acc2_u = seg2(tl2u, src_um, dst_um, tl2m, src_mu, dst_mu, zrow)

    hm2, hu2 = _tc3(acc2_m, acc2_u, zmx, zux)

    dec = _make_decoder()
    ug, mg = dec(hu2, hm2, e_u, e_m)
    return _tcdot(ug, mg).reshape(LBL)


# speedup vs baseline: 5.8617x; 1.2412x over previous
import functools

import jax
import jax.numpy as jnp
from jax import lax
from jax.experimental import pallas as pl
from jax.experimental.pallas import tpu as pltpu
from jax.experimental.pallas import tpu_sc as plsc

N = 10000
NPAD = 10240
D = 128
H = 64
W = 80
E = 320000
LBL = 100000

NC = 2
NS = 16
NW = NC * NS
LANE = 16

EPT = E // NW
K = 400
NCHUNK = EPT // K
SCALE = 64.0
RPT = NPAD // NS

K2 = 400
NCH_DEC = LBL // K2
DEC_PER_TILE = (NCH_DEC + NW - 1) // NW


def _sc_mesh():
    return plsc.VectorSubcoreMesh(core_axis_name="c", subcore_axis_name="s",
                                  num_cores=NC, num_subcores=NS)


def _make_segsum():
    out_type = [jax.ShapeDtypeStruct((NC, NPAD, W), jnp.float32),
                jax.ShapeDtypeStruct((NC, NPAD, W), jnp.float32)]
    scratch = [
        pltpu.VMEM((EPT,), jnp.int32),
        pltpu.VMEM((NCHUNK, K), jnp.int32),
        pltpu.VMEM((K, W), jnp.float32),
        pltpu.VMEM_SHARED((NPAD, W), jnp.float32),
        pltpu.SemaphoreType.DMA,
    ]

    def body(t_a, src_a, dst_a, t_b, src_b, dst_b, zrow,
             acc_a_o, acc_b_o, idx_s, idx_d, rows_v, acc_sh, sem):
        cid = lax.axis_index("c")
        sid = lax.axis_index("s")
        wid = sid * NC + cid
        rbase = pl.multiple_of(sid * RPT, 8)

        def one_type(tbl, src, dst, acc_o):
            pltpu.sync_copy(
                src.at[pl.ds(pl.multiple_of(wid * EPT, 8), EPT)], idx_s)
            pltpu.sync_copy(dst.at[wid], idx_d)
            pltpu.sync_copy(zrow.at[pl.ds(rbase, RPT)],
                            acc_sh.at[pl.ds(rbase, RPT)])
            plsc.subcore_barrier()

            def chunk(ci, carry):
                sidx = idx_s.at[pl.ds(pl.multiple_of(ci * K, 8), K)]
                pltpu.async_copy(tbl.at[sidx], rows_v, sem).wait()
                pltpu.sync_copy(rows_v, acc_sh.at[idx_d.at[ci]], add=True)
                return carry

            lax.fori_loop(0, NCHUNK, chunk, 0)
            plsc.subcore_barrier()

            pltpu.sync_copy(acc_sh.at[pl.ds(rbase, RPT)],
                            acc_o.at[cid, pl.ds(rbase, RPT)])
            plsc.subcore_barrier()

        one_type(t_a, src_a, dst_a, acc_a_o)
        one_type(t_b, src_b, dst_b, acc_b_o)

    return pl.kernel(body, out_type=out_type, mesh=_sc_mesh(),
                     scratch_types=scratch,
                     compiler_params=pltpu.CompilerParams(
                         use_tc_tiling_on_sc=False))


def _make_decoder():
    out_type = [jax.ShapeDtypeStruct((LBL, W), jnp.float32),
                jax.ShapeDtypeStruct((LBL, W), jnp.float32)]
    scratch = [
        pltpu.VMEM((K2,), jnp.int32),
        pltpu.VMEM((K2,), jnp.int32),
        pltpu.VMEM((K2, W), jnp.float32),
        pltpu.VMEM((K2, W), jnp.float32),
        pltpu.SemaphoreType.DMA,
    ]

    def body(h_u, h_m, e_u, e_m, ug_o, mg_o, idx_u, idx_m, u_rows, m_rows,
             sem):
        cid = lax.axis_index("c")
        sid = lax.axis_index("s")
        wid = sid * NC + cid

        def per_j(j, carry):
            ch = j * NW + wid

            @pl.when(ch < NCH_DEC)
            def _():
                base = pl.multiple_of(ch * K2, 8)
                pltpu.sync_copy(e_u.at[pl.ds(base, K2)], idx_u)
                pltpu.sync_copy(e_m.at[pl.ds(base, K2)], idx_m)
                pltpu.async_copy(h_u.at[idx_u], u_rows, sem).wait()
                pltpu.async_copy(h_m.at[idx_m], m_rows, sem).wait()
                pltpu.sync_copy(u_rows, ug_o.at[pl.ds(base, K2)])
                pltpu.sync_copy(m_rows, mg_o.at[pl.ds(base, K2)])

            return carry

        lax.fori_loop(0, DEC_PER_TILE, per_j, 0)

    return pl.kernel(body, out_type=out_type, mesh=_sc_mesh(),
                     scratch_types=scratch,
                     compiler_params=pltpu.CompilerParams(
                         use_tc_tiling_on_sc=False))


def _tcdot_body(ug_r, mg_r, out_r):
    u = ug_r[...]
    m = mg_r[...]
    out_r[...] = jnp.sum(u[:, :H] * m[:, :H], axis=1, keepdims=True)


def _tcdot(ug, mg):
    RL = 2000
    return pl.pallas_call(
        _tcdot_body,
        grid=(LBL // RL,),
        in_specs=[pl.BlockSpec((RL, W), lambda i: (i, 0)),
                  pl.BlockSpec((RL, W), lambda i: (i, 0))],
        out_specs=pl.BlockSpec((RL, 1), lambda i: (i, 0)),
        out_shape=jax.ShapeDtypeStruct((LBL, 1), jnp.float32),
    )(ug, mg)


R = 1000
G = N // R

_mm = functools.partial(jnp.dot, precision="highest",
                        preferred_element_type=jnp.float32)


def _with_cols(main, extra_col=None):
    cols = [main]
    used = H
    if extra_col is not None:
        cols.append(extra_col)
        used += 1
    cols.append(jnp.zeros((main.shape[0], W - used), jnp.float32))
    return jnp.concatenate(cols, axis=1)


def _quant(rows_f32):
    return jnp.clip(jnp.round(rows_f32 * SCALE), -32768.0,
                    32767.0).astype(jnp.int16)


def _tc1_body(xu_r, xm_r, wlu_r, wru_r, wlm_r, wrm_r,
              tlu_r, yru_r, tlm_r, yrm_r):
    xu = xu_r[...]
    xm = xm_r[...]
    one = jnp.ones((R, 1), jnp.float32)
    tlu_r[...] = _with_cols(_mm(xu, wlu_r[...]), one)
    yru_r[...] = _mm(xu, wru_r[...])
    tlm_r[...] = _with_cols(_mm(xm, wlm_r[...]), one)
    yrm_r[...] = _mm(xm, wrm_r[...])


def _tc1(xu, xm, wl1_um, wr1_mu, wl1_mu, wr1_um):
    fs = jnp.float32
    return pl.pallas_call(
        _tc1_body,
        grid=(G,),
        in_specs=[
            pl.BlockSpec((R, D), lambda i: (i, 0)),
            pl.BlockSpec((R, D), lambda i: (i, 0)),
            pl.BlockSpec((D, H), lambda i: (0, 0)),
            pl.BlockSpec((D, H), lambda i: (0, 0)),
            pl.BlockSpec((D, H), lambda i: (0, 0)),
            pl.BlockSpec((D, H), lambda i: (0, 0)),
        ],
        out_specs=[
            pl.BlockSpec((R, W), lambda i: (i, 0)),
            pl.BlockSpec((R, H), lambda i: (i, 0)),
            pl.BlockSpec((R, W), lambda i: (i, 0)),
            pl.BlockSpec((R, H), lambda i: (i, 0)),
        ],
        out_shape=[
            jax.ShapeDtypeStruct((N, W), fs),
            jax.ShapeDtypeStruct((N, H), fs),
            jax.ShapeDtypeStruct((N, W), fs),
            jax.ShapeDtypeStruct((N, H), fs),
        ],
    )(xu, xm, wl1_um, wr1_mu, wl1_mu, wr1_um)


def _tc2_body(am_r, au_r, yrm_r, yru_r, b1um_r, b1mu_r,
              wl2um_r, wr2um_r, wl2mu_r, wr2mu_r, b2um_r, b2mu_r,
              tl2u_r, tl2m_r, zmx_r, zux_r):
    am = am_r[...]
    au = au_r[...]
    sm = am[0] + am[1]
    su = au[0] + au[1]
    icm = 1.0 / jnp.maximum(sm[:, H:H + 1], 1.0)
    icu = 1.0 / jnp.maximum(su[:, H:H + 1], 1.0)
    hm = jnp.maximum(sm[:, :H] * icm + yrm_r[...] + b1um_r[...], 0.0)
    hu = jnp.maximum(su[:, :H] * icu + yru_r[...] + b1mu_r[...], 0.0)
    tl2u_r[...] = _with_cols(_mm(hu, wl2um_r[...]))
    tl2m_r[...] = _with_cols(_mm(hm, wl2mu_r[...]))
    zmx_r[...] = _with_cols(_mm(hm, wr2um_r[...]) + b2um_r[...], icm)
    zux_r[...] = _with_cols(_mm(hu, wr2mu_r[...]) + b2mu_r[...], icu)


def _tc2(acc_m, acc_u, yrm, yru, b1um, b1mu,
         wl2um, wr2um, wl2mu, wr2mu, b2um, b2mu):
    fs = jnp.float32
    part = pl.BlockSpec((NC, R, W), lambda i: (0, i, 0))
    row = pl.BlockSpec((R, H), lambda i: (i, 0))
    roww = pl.BlockSpec((R, W), lambda i: (i, 0))
    w = pl.BlockSpec((H, H), lambda i: (0, 0))
    b = pl.BlockSpec((1, H), lambda i: (0, 0))
    return pl.pallas_call(
        _tc2_body,
        grid=(G,),
        in_specs=[part, part, row, row, b, b, w, w, w, w, b, b],
        out_specs=[roww] * 4,
        out_shape=[jax.ShapeDtypeStruct((N, W), fs)] * 4,
    )(acc_m, acc_u, yrm, yru, b1um, b1mu,
      wl2um, wr2um, wl2mu, wr2mu, b2um, b2mu)


def _tc3_body(am_r, au_r, zmx_r, zux_r, hm2_r, hu2_r):
    am = am_r[...]
    au = au_r[...]
    sm = am[0] + am[1]
    su = au[0] + au[1]
    zmx = zmx_r[...]
    zux = zux_r[...]
    hm2 = sm[:, :H] * zmx[:, H:H + 1] + zmx[:, :H]
    hu2 = su[:, :H] * zux[:, H:H + 1] + zux[:, :H]
    hm2_r[...] = _with_cols(hm2)
    hu2_r[...] = _with_cols(hu2)


def _tc3(acc2_m, acc2_u, zmx, zux):
    fs = jnp.float32
    part = pl.BlockSpec((NC, R, W), lambda i: (0, i, 0))
    roww = pl.BlockSpec((R, W), lambda i: (i, 0))
    return pl.pallas_call(
        _tc3_body,
        grid=(G,),
        in_specs=[part, part, roww, roww],
        out_specs=[roww] * 2,
        out_shape=[jax.ShapeDtypeStruct((N, W), fs)] * 2,
    )(acc2_m, acc2_u, zmx, zux)


def kernel(x_user, x_movie, edge_index_um, edge_index_mu, edge_label_index,
           Wl1_um, Wr1_um, b1_um, Wl1_mu, Wr1_mu, b1_mu,
           Wl2_um, Wr2_um, b2_um, Wl2_mu, Wr2_mu, b2_mu):
    src_um = edge_index_um[0]
    dst_um = edge_index_um[1].reshape(NW, NCHUNK, K)
    src_mu = edge_index_mu[0]
    dst_mu = edge_index_mu[1].reshape(NW, NCHUNK, K)
    e_u, e_m = edge_label_index[0], edge_label_index[1]

    zrow = jnp.zeros((NPAD, W), jnp.float32)

    b1um = b1_um.reshape(1, H)
    b1mu = b1_mu.reshape(1, H)
    b2um = b2_um.reshape(1, H)
    b2mu = b2_mu.reshape(1, H)

    tlu, yru, tlm, yrm = _tc1(x_user, x_movie, Wl1_um, Wr1_mu, Wl1_mu,
                              Wr1_um)

    seg = _make_segsum()
    acc_m, acc_u = seg(tlu, src_um, dst_um, tlm, src_mu, dst_mu, zrow)

    tl2u, tl2m, zmx, zux = _tc2(acc_m, acc_u, yrm, yru, b1um, b1mu,
                                Wl2_um, Wr2_um, Wl2_mu, Wr2_mu, b2um, b2mu)

    seg2 = _make_segsum()
    acc2_m, acc2_u = seg2(tl2u, src_um, dst_um, tl2m, src_mu, dst_mu, zrow)

    hm2, hu2 = _tc3(acc2_m, acc2_u, zmx, zux)

    dec = _make_decoder()
    ug, mg = dec(hu2, hm2, e_u, e_m)
    return _tcdot(ug, mg).reshape(LBL)
